# Initial kernel scaffold; baseline (speedup 1.0000x reference)
import jax
import jax.numpy as jnp
from jax.experimental import pallas as pl


def kernel(x, table):
    raise NotImplementedError("write your pallas kernel here")


# speedup vs baseline: 3.8078x; 3.8078x over previous
import jax
import jax.numpy as jnp
from jax.experimental import pallas as pl

_HIDDEN = 1024
_NUM_EMB = 3
_CHUNK = 512


def _emb_body(x_ref, t_ref, o_ref):
    xc = x_ref[0, 0, :][:, None]
    r1 = t_ref[1, :][None, :]
    r2 = t_ref[2, :][None, :]
    w1 = (xc == 1).astype(jnp.float32)
    w2 = (xc == 2).astype(jnp.float32)
    o_ref[...] = w1 * r1 + w2 * r2


def kernel(x, table):
    b, s = x.shape
    n = b * s
    grid = n // _CHUNK
    x_r = x.reshape(grid, 1, _CHUNK).astype(jnp.int32)
    out = pl.pallas_call(
        _emb_body,
        grid=(grid,),
        in_specs=[
            pl.BlockSpec((1, 1, _CHUNK), lambda i: (i, 0, 0)),
            pl.BlockSpec((_NUM_EMB, _HIDDEN), lambda i: (0, 0)),
        ],
        out_specs=pl.BlockSpec((_CHUNK, _HIDDEN), lambda i: (i, 0)),
        out_shape=jax.ShapeDtypeStruct((n, _HIDDEN), jnp.float32),
    )(x_r, table)
    return out.reshape(b, s, _HIDDEN)
